# Initial kernel scaffold; baseline (speedup 1.0000x reference)
import functools
import math

import jax
import jax.numpy as jnp
from jax import lax
from jax.experimental import pallas as pl
from jax.experimental.pallas import tpu as pltpu
from jax.experimental.pallas import tpu_sc as plsc

N = 10000
E = 320000
D = 128
ED = 16
BN_EPS = 1e-5

NC = 2
NS = 16
LANE = 16

G = 128
ROWS = E // G
ROWS_PER_SC = ROWS // NC
RPT = ROWS_PER_SC // NS
RPT_REM = ROWS_PER_SC - RPT * NS
NZR = N // NS


def _sc_aggregate(x, src_rows, dst_rows, ep):
  mesh = plsc.VectorSubcoreMesh(core_axis_name="c", subcore_axis_name="s")

  @functools.partial(
      pl.kernel,
      out_type=jax.ShapeDtypeStruct((NC, N, D), jnp.float32),
      mesh=mesh,
      scratch_types=[
          pltpu.VMEM_SHARED((N, D), jnp.float32),
          pltpu.VMEM((1, G), jnp.int32),
          pltpu.VMEM((1, G), jnp.int32),
          pltpu.VMEM((G, D), jnp.float32),
          pltpu.VMEM((G, D), jnp.float32),
          pltpu.SemaphoreType.DMA,
      ],
  )
  def agg_kernel(x_hbm, src_hbm, dst_hbm, ep_hbm, out_hbm,
                 aggr_sh, src_i, dst_i, ep_b, rows_b, sem):
    c = lax.axis_index("c")
    s = lax.axis_index("s")

    def _zrow(r, carry):
      for k in range(D // LANE):
        rows_b[r, pl.ds(k * LANE, LANE)] = jnp.zeros((LANE,), jnp.float32)
      return carry
    lax.fori_loop(0, G, _zrow, 0)
    z0 = s * NZR
    nfull = NZR // G
    for q in range(nfull):
      pltpu.sync_copy(rows_b, aggr_sh.at[pl.ds(z0 + q * G, G)])
    rem = NZR - nfull * G
    if rem:
      pltpu.sync_copy(rows_b.at[pl.ds(0, rem)],
                      aggr_sh.at[pl.ds(z0 + nfull * G, rem)])
    plsc.subcore_barrier()

    r0 = c * ROWS_PER_SC + s * RPT + jnp.minimum(s, RPT_REM)
    nr = RPT + jnp.where(s < RPT_REM, 1, 0)

    def _chunk(i, carry):
      r = r0 + i
      pltpu.sync_copy(src_hbm.at[pl.ds(r, 1)], src_i)
      pltpu.sync_copy(dst_hbm.at[pl.ds(r, 1)], dst_i)
      pltpu.sync_copy(ep_hbm.at[pl.ds(r * G, G)], ep_b)
      pltpu.async_copy(x_hbm.at[src_i.at[0]], rows_b, sem).wait()

      def _crow(rr, inner):
        for k in range(D // LANE):
          sl = pl.ds(k * LANE, LANE)
          rows_b[rr, sl] = jnp.maximum(rows_b[rr, sl] + ep_b[rr, sl], 0.0)
        return inner
      lax.fori_loop(0, G, _crow, 0)

      pltpu.sync_copy(rows_b, aggr_sh.at[dst_i.at[0]], add=True)
      return carry
    lax.fori_loop(0, nr, _chunk, 0)

    plsc.subcore_barrier()
    pltpu.sync_copy(aggr_sh.at[pl.ds(z0, NZR)],
                    out_hbm.at[c, pl.ds(z0, NZR)])

  return agg_kernel(x, src_rows, dst_rows, ep)


def _edge_proj(edge_attr, We, be):
  BE = 2000

  def body(ea_ref, we_ref, be_ref, out_ref):
    out_ref[...] = (
        jnp.dot(ea_ref[...], we_ref[...], preferred_element_type=jnp.float32)
        + be_ref[...])

  return pl.pallas_call(
      body,
      grid=(E // BE,),
      in_specs=[
          pl.BlockSpec((BE, ED), lambda i: (i, 0)),
          pl.BlockSpec((ED, D), lambda i: (0, 0)),
          pl.BlockSpec((1, D), lambda i: (0, 0)),
      ],
      out_specs=pl.BlockSpec((BE, D), lambda i: (i, 0)),
      out_shape=jax.ShapeDtypeStruct((E, D), jnp.float32),
  )(edge_attr, We, be.reshape(1, D))


def _node_update(x, aggr, Wp, bp):
  BN = 1000

  def body(x_ref, a_ref, w_ref, b_ref, out_ref):
    y = x_ref[...] + a_ref[0] + a_ref[1]
    out_ref[...] = jnp.maximum(
        jnp.dot(y, w_ref[...], preferred_element_type=jnp.float32)
        + b_ref[...], 0.0)

  return pl.pallas_call(
      body,
      grid=(N // BN,),
      in_specs=[
          pl.BlockSpec((BN, D), lambda i: (i, 0)),
          pl.BlockSpec((NC, BN, D), lambda i: (0, i, 0)),
          pl.BlockSpec((D, D), lambda i: (0, 0)),
          pl.BlockSpec((1, D), lambda i: (0, 0)),
      ],
      out_specs=pl.BlockSpec((BN, D), lambda i: (i, 0)),
      out_shape=jax.ShapeDtypeStruct((N, D), jnp.float32),
  )(x, aggr, Wp, bp.reshape(1, D))


def kernel(x, edge_index, edge_attr,
           We0, be0, W0, b0, g0, bt0,
           We1, be1, W1, b1, g1, bt1,
           We2, be2, W2, b2, g2, bt2):
  scale = 1.0 / math.sqrt(1.0 + BN_EPS)
  src_rows = edge_index[0].reshape(ROWS, G)
  dst_rows = edge_index[1].reshape(ROWS, G)

  h = x
  for We, be, W, b, g, bt in (
      (We0, be0, W0, b0, g0, bt0),
      (We1, be1, W1, b1, g1, bt1),
      (We2, be2, W2, b2, g2, bt2)):
    ep = _edge_proj(edge_attr, We, be)
    aggr = _sc_aggregate(h, src_rows, dst_rows, ep)
    gs = g * scale
    h = _node_update(h, aggr, W * gs[None, :], b * gs + bt)
  return h


# speedup vs baseline: 2.8993x; 2.8993x over previous
import functools
import math

import jax
import jax.numpy as jnp
from jax import lax
from jax.experimental import pallas as pl
from jax.experimental.pallas import tpu as pltpu
from jax.experimental.pallas import tpu_sc as plsc

N = 10000
E = 320000
D = 128
ED = 16
BN_EPS = 1e-5

NC = 2
NS = 16
LANE = 16

G = 128
ROWS = E // G
ROWS_PER_SC = ROWS // NC
RPT = ROWS_PER_SC // NS
RPT_REM = ROWS_PER_SC - RPT * NS
NZR = N // NS


def _sc_aggregate(x, src_rows, dst_rows, ep):
  mesh = plsc.VectorSubcoreMesh(core_axis_name="c", subcore_axis_name="s")

  @functools.partial(
      pl.kernel,
      out_type=jax.ShapeDtypeStruct((NC, N, D), jnp.float32),
      mesh=mesh,
      compiler_params=pltpu.CompilerParams(use_tc_tiling_on_sc=False),
      scratch_types=[
          pltpu.VMEM_SHARED((N, D), jnp.float32),
          pltpu.VMEM((1, G), jnp.int32),
          pltpu.VMEM((1, G), jnp.int32),
          pltpu.VMEM((G, D), jnp.float32),
          pltpu.VMEM((G, D), jnp.float32),
          pltpu.SemaphoreType.DMA,
      ],
  )
  def agg_kernel(x_hbm, src_hbm, dst_hbm, ep_hbm, out_hbm,
                 aggr_sh, src_i, dst_i, ep_b, rows_b, sem):
    c = lax.axis_index("c")
    s = lax.axis_index("s")

    def _zrow(r, carry):
      for k in range(D // LANE):
        rows_b[r, pl.ds(k * LANE, LANE)] = jnp.zeros((LANE,), jnp.float32)
      return carry
    lax.fori_loop(0, G, _zrow, 0)
    z0 = s * NZR
    nfull = NZR // G
    for q in range(nfull):
      pltpu.sync_copy(rows_b, aggr_sh.at[pl.ds(z0 + q * G, G)])
    rem = NZR - nfull * G
    if rem:
      pltpu.sync_copy(rows_b.at[pl.ds(0, rem)],
                      aggr_sh.at[pl.ds(z0 + nfull * G, rem)])
    plsc.subcore_barrier()

    r0 = c * ROWS_PER_SC + s * RPT + jnp.minimum(s, RPT_REM)
    nr = RPT + jnp.where(s < RPT_REM, 1, 0)

    def _chunk(i, carry):
      r = r0 + i
      pltpu.sync_copy(src_hbm.at[pl.ds(r, 1)], src_i)
      pltpu.sync_copy(dst_hbm.at[pl.ds(r, 1)], dst_i)
      pltpu.sync_copy(ep_hbm.at[pl.ds(r * G, G)], ep_b)
      pltpu.async_copy(x_hbm.at[src_i.at[0]], rows_b, sem).wait()

      def _crow(rr, inner):
        for k in range(D // LANE):
          sl = pl.ds(k * LANE, LANE)
          rows_b[rr, sl] = jnp.maximum(rows_b[rr, sl] + ep_b[rr, sl], 0.0)
        return inner
      lax.fori_loop(0, G, _crow, 0)

      pltpu.sync_copy(rows_b, aggr_sh.at[dst_i.at[0]], add=True)
      return carry
    lax.fori_loop(0, nr, _chunk, 0)

    plsc.subcore_barrier()
    pltpu.sync_copy(aggr_sh.at[pl.ds(z0, NZR)],
                    out_hbm.at[c, pl.ds(z0, NZR)])

  return agg_kernel(x, src_rows, dst_rows, ep)


def _edge_proj(edge_attr, We, be):
  BE = 2000

  def body(ea_ref, we_ref, be_ref, out_ref):
    out_ref[...] = (
        jnp.dot(ea_ref[...], we_ref[...], preferred_element_type=jnp.float32)
        + be_ref[...])

  return pl.pallas_call(
      body,
      grid=(E // BE,),
      in_specs=[
          pl.BlockSpec((BE, ED), lambda i: (i, 0)),
          pl.BlockSpec((ED, D), lambda i: (0, 0)),
          pl.BlockSpec((1, D), lambda i: (0, 0)),
      ],
      out_specs=pl.BlockSpec((BE, D), lambda i: (i, 0)),
      out_shape=jax.ShapeDtypeStruct((E, D), jnp.float32),
  )(edge_attr, We, be.reshape(1, D))


def _node_update(x, aggr, Wp, bp):
  BN = 1000

  def body(x_ref, a_ref, w_ref, b_ref, out_ref):
    y = x_ref[...] + a_ref[0] + a_ref[1]
    out_ref[...] = jnp.maximum(
        jnp.dot(y, w_ref[...], preferred_element_type=jnp.float32)
        + b_ref[...], 0.0)

  return pl.pallas_call(
      body,
      grid=(N // BN,),
      in_specs=[
          pl.BlockSpec((BN, D), lambda i: (i, 0)),
          pl.BlockSpec((NC, BN, D), lambda i: (0, i, 0)),
          pl.BlockSpec((D, D), lambda i: (0, 0)),
          pl.BlockSpec((1, D), lambda i: (0, 0)),
      ],
      out_specs=pl.BlockSpec((BN, D), lambda i: (i, 0)),
      out_shape=jax.ShapeDtypeStruct((N, D), jnp.float32),
  )(x, aggr, Wp, bp.reshape(1, D))


def kernel(x, edge_index, edge_attr,
           We0, be0, W0, b0, g0, bt0,
           We1, be1, W1, b1, g1, bt1,
           We2, be2, W2, b2, g2, bt2):
  scale = 1.0 / math.sqrt(1.0 + BN_EPS)
  src_rows = edge_index[0].reshape(ROWS, G)
  dst_rows = edge_index[1].reshape(ROWS, G)

  h = x
  for We, be, W, b, g, bt in (
      (We0, be0, W0, b0, g0, bt0),
      (We1, be1, W1, b1, g1, bt1),
      (We2, be2, W2, b2, g2, bt2)):
    ep = _edge_proj(edge_attr, We, be)
    aggr = _sc_aggregate(h, src_rows, dst_rows, ep)
    gs = g * scale
    h = _node_update(h, aggr, W * gs[None, :], b * gs + bt)
  return h
